# baseline (speedup 1.0000x reference)
import functools

import jax
import jax.numpy as jnp
import numpy as np
from jax import lax
from jax.experimental import pallas as pl
from jax.experimental.pallas import tpu as pltpu
from jax.experimental.pallas import tpu_sc as plsc

F = 256
DW = 128
ND = 30000
C = 2560
STR = 168
CP = 2688
NCH = 6
HALF = NCH * C
NDP = 2 * HALF
K = 16
NSUB = 16
E1 = 256000
E2 = 64000
PK = 13
EPK = 16


def _prefix16(mi, iota, zero):
    cs = mi
    for b in (1, 2, 4, 8):
        cs = cs + jnp.where(iota >= b, jnp.take(cs, jnp.maximum(iota - b, 0)),
                            zero)
    return cs


def _compact16(v, sh, iota):
    for b in (1, 2, 4, 8):
        idx = jnp.minimum(iota + b, 15)
        cv = jnp.take(v, idx)
        csh = jnp.take(sh, idx)
        mv = (csh & b) != 0
        v = jnp.where(mv, cv, v)
        sh = jnp.where(mv, csh, sh)
    return v


def _make_sc_agg(E):
    Ew = E // NSUB
    NF = Ew // 16
    SEL = Ew + K + 16
    mesh = plsc.VectorSubcoreMesh(core_axis_name="c", subcore_axis_name="s")

    @functools.partial(
        pl.kernel,
        out_type=(jax.ShapeDtypeStruct((NDP, 2, DW), jnp.float32),
                  jax.ShapeDtypeStruct((NDP, DW), jnp.float32)),
        mesh=mesh,
        scratch_types=[
            pltpu.VMEM((Ew + 16,), jnp.int32),
            pltpu.VMEM((SEL,), jnp.int32),
            pltpu.VMEM((K,), jnp.int32),
            pltpu.VMEM((K,), jnp.int32),
            pltpu.VMEM((K, 2, DW), jnp.float32),
            pltpu.VMEM((K, DW), jnp.float32),
            pltpu.VMEM_SHARED((CP, 2, DW), jnp.float32),
            pltpu.VMEM_SHARED((CP, DW), jnp.float32),
            pltpu.SemaphoreType.DMA,
        ],
    )
    def sc_agg(ed_hbm, h_hbm, zf_hbm, zd_hbm, ones_hbm,
               agg_hbm, deg_hbm,
               ed, sel, idxs, idxd, rows, ones_v, agg_s, deg_s, sem):
        c = lax.axis_index("c")
        s = lax.axis_index("s")
        pltpu.sync_copy(ed_hbm.at[pl.ds(s * Ew, Ew)], ed.at[pl.ds(0, Ew)])
        pltpu.sync_copy(ones_hbm, ones_v)

        iota = lax.iota(jnp.int32, 16)
        one = jnp.full((16,), 1, jnp.int32)
        zero = jnp.full((16,), 0, jnp.int32)
        garb = jnp.full((16,), C, jnp.int32)
        egarb = jnp.full((16,), 65535, jnp.int32)
        ar1 = iota + 1

        hlo = c * HALF

        def pfbody(i, cnt):
            pk = ed[pl.ds(i * 16, 16)]
            d = pk & (2 ** EPK - 1)
            m = (d >= hlo) & (d < hlo + HALF)
            mi = jnp.where(m, one, zero)
            cs = _prefix16(mi, iota, zero)
            sh = jnp.where(m, ar1 - cs, zero)
            v = _compact16(jnp.where(m, pk, egarb), sh, iota)
            ed[pl.ds(cnt, 16)] = v
            return cnt + cs[15]

        cnt0 = lax.fori_loop(0, NF, pfbody, jnp.int32(0), unroll=2)
        ed[pl.ds(cnt0, 16)] = egarb
        nf0 = (cnt0 + 15) // 16

        for j in range(NCH):
            lo = hlo + j * C
            pltpu.sync_copy(zf_hbm, agg_s.at[pl.ds(s * STR, STR)])
            pltpu.sync_copy(zd_hbm, deg_s.at[pl.ds(s * STR, STR)])
            plsc.subcore_barrier()

            def fbody(i, cnt):
                pk = ed[pl.ds(i * 16, 16)]
                d = pk & (2 ** EPK - 1)
                m = (d >= lo) & (d < lo + C)
                mi = jnp.where(m, one, zero)
                cs = _prefix16(mi, iota, zero)
                sh = jnp.where(m, ar1 - cs, zero)
                sv = lax.shift_right_logical(pk, EPK)
                pkc = jnp.where(m, lax.shift_left(sv, PK) + (d - lo), garb)
                sel[pl.ds(cnt, 16)] = _compact16(pkc, sh, iota)
                return cnt + cs[15]

            cnt = lax.fori_loop(0, nf0, fbody, jnp.int32(0))

            def pbody(u, acc):
                sel[pl.ds(cnt + u * 16, 16)] = garb
                return acc

            lax.fori_loop(0, K // 16, pbody, jnp.int32(0))

            def bbody(b, acc):
                for u in range(K // 16):
                    pkv = sel[pl.ds(b * K + u * 16, 16)]
                    idxs[pl.ds(u * 16, 16)] = lax.shift_right_logical(pkv, PK)
                    idxd[pl.ds(u * 16, 16)] = pkv & (2 ** PK - 1)

                pltpu.async_copy(h_hbm.at[idxs], rows, sem).wait()
                pltpu.sync_copy(rows, agg_s.at[idxd], add=True)
                pltpu.sync_copy(ones_v, deg_s.at[idxd], add=True)
                return acc

            lax.fori_loop(0, (cnt + (K - 1)) // K, bbody, jnp.int32(0))
            plsc.subcore_barrier()

            TAIL = C - (NSUB - 1) * STR

            @pl.when(s < NSUB - 1)
            def _wb():
                pltpu.sync_copy(agg_s.at[pl.ds(s * STR, STR)],
                                agg_hbm.at[pl.ds(lo + s * STR, STR)])
                pltpu.sync_copy(deg_s.at[pl.ds(s * STR, STR)],
                                deg_hbm.at[pl.ds(lo + s * STR, STR)])

            @pl.when(s == NSUB - 1)
            def _wbt():
                pltpu.sync_copy(agg_s.at[pl.ds((NSUB - 1) * STR, TAIL)],
                                agg_hbm.at[pl.ds(lo + (NSUB - 1) * STR, TAIL)])
                pltpu.sync_copy(deg_s.at[pl.ds((NSUB - 1) * STR, TAIL)],
                                deg_hbm.at[pl.ds(lo + (NSUB - 1) * STR, TAIL)])

    return sc_agg


_sc_agg_1 = _make_sc_agg(E1)
_sc_agg_2 = _make_sc_agg(E2)

BT = 1000


def _tc_body(agg_ref, deg_ref, hdst_ref, m_ref, w_ref, b_ref, out_ref):
    deg = jnp.maximum(deg_ref[:, 0:1], 1.0)
    aggn = agg_ref[...] / deg
    wa = w_ref[:, 0:F]
    wh = w_ref[:, F:2 * F]
    dn = (((1,), (1,)), ((), ()))
    acc = lax.dot_general(aggn, wa, dn, preferred_element_type=jnp.float32,
                          precision=lax.Precision.HIGHEST)
    acc = acc + lax.dot_general(hdst_ref[...], wh, dn,
                                preferred_element_type=jnp.float32,
                                precision=lax.Precision.HIGHEST)
    acc = jnp.maximum(acc + b_ref[...], 0.0)
    out_ref[...] = jnp.where(m_ref[...] != 0, acc * 2.0, 0.0)


def _tc_layer(agg, deg, hdst, mask_i8, W, b):
    return pl.pallas_call(
        _tc_body,
        grid=(ND // BT,),
        in_specs=[
            pl.BlockSpec((BT, F), lambda i: (i, 0)),
            pl.BlockSpec((BT, DW), lambda i: (i, 0)),
            pl.BlockSpec((BT, F), lambda i: (i, 0)),
            pl.BlockSpec((BT, F), lambda i: (i, 0)),
            pl.BlockSpec((F, 2 * F), lambda i: (0, 0)),
            pl.BlockSpec((1, F), lambda i: (0, 0)),
        ],
        out_specs=pl.BlockSpec((BT, F), lambda i: (i, 0)),
        out_shape=jax.ShapeDtypeStruct((ND, F), jnp.float32),
    )(agg, deg, hdst, mask_i8, W, b)


def _rotl(x, r):
    return ((x << np.uint32(r)) | (x >> np.uint32(32 - r))).astype(np.uint32)


def _threefry2x32(k0, k1, x0, x1):
    rot = ((13, 15, 26, 6), (17, 29, 16, 24))
    ks = (np.uint32(k0), np.uint32(k1),
          np.uint32(k0) ^ np.uint32(k1) ^ np.uint32(0x1BD11BDA))
    x0 = (x0 + ks[0]).astype(np.uint32)
    x1 = (x1 + ks[1]).astype(np.uint32)
    for i in range(5):
        for r in rot[i % 2]:
            x0 = (x0 + x1).astype(np.uint32)
            x1 = _rotl(x1, r)
            x1 = x1 ^ x0
        x0 = (x0 + ks[(i + 1) % 3]).astype(np.uint32)
        x1 = (x1 + ks[(i + 2) % 3] + np.uint32(i + 1)).astype(np.uint32)
    return x0, x1


def _np_split(k0, k1):
    a, b = _threefry2x32(k0, k1, np.zeros(2, np.uint32),
                         np.arange(2, dtype=np.uint32))
    return (a[0], b[0]), (a[1], b[1])


def _np_bernoulli_half(k0, k1, shape):
    n = int(np.prod(shape))
    a, b = _threefry2x32(k0, k1, np.zeros(n, np.uint32),
                         np.arange(n, dtype=np.uint32))
    w = a ^ b
    fl = (((w >> np.uint32(9)) | np.uint32(0x3F800000)).view(np.float32)
          - np.float32(1.0))
    return (fl < np.float32(0.5)).reshape(shape)


def _dropout_masks():
    knew, sub = _np_split(np.uint32(0), np.uint32(42))
    m1 = _np_bernoulli_half(sub[0], sub[1], (50000, F))[:ND]
    _, sub2 = _np_split(*knew)
    m2 = _np_bernoulli_half(sub2[0], sub2[1], (ND, F))
    return m1.astype(np.int8), m2.astype(np.int8)


_M1, _M2 = _dropout_masks()


def kernel(x, edge, num_sampled_nodes, num_sampled_edges, W1, b1, W2, b2):
    del num_sampled_nodes, num_sampled_edges
    edge = edge.astype(jnp.int32)
    ed1 = (edge[E2:, 0] << EPK) | edge[E2:, 1]
    ed2 = (edge[:E2, 0] << EPK) | edge[:E2, 1]

    m1, m2 = _M1, _M2

    zf = jnp.zeros((STR, 2, DW), jnp.float32)
    zd = jnp.zeros((STR, DW), jnp.float32)
    ones = jnp.ones((K, DW), jnp.float32)

    agg1, deg1 = _sc_agg_1(ed1, x.reshape(-1, 2, DW), zf, zd, ones)
    agg1 = agg1.reshape(-1, F)
    h1 = _tc_layer(agg1, deg1, x, m1, W1, b1.reshape(1, F))
    agg2, deg2 = _sc_agg_2(ed2, h1.reshape(-1, 2, DW), zf, zd, ones)
    agg2 = agg2.reshape(-1, F)
    return _tc_layer(agg2, deg2, h1, m2, W2, b2.reshape(1, F))


# speedup vs baseline: 1.0370x; 1.0370x over previous
import functools

import jax
import jax.numpy as jnp
import numpy as np
from jax import lax
from jax.experimental import pallas as pl
from jax.experimental.pallas import tpu as pltpu
from jax.experimental.pallas import tpu_sc as plsc

F = 256
DW = 128
ND = 30000
C = 2560
STR = 168
CP = 2688
NCH = 6
HALF = NCH * C
NDP = 2 * HALF
K = 32
NSUB = 16
E1 = 256000
E2 = 64000
PK = 13
EPK = 16


def _prefix16(mi, iota, zero):
    cs = mi
    for b in (1, 2, 4, 8):
        cs = cs + jnp.where(iota >= b, jnp.take(cs, jnp.maximum(iota - b, 0)),
                            zero)
    return cs


def _compact16(v, sh, iota):
    for b in (1, 2, 4, 8):
        idx = jnp.minimum(iota + b, 15)
        cv = jnp.take(v, idx)
        csh = jnp.take(sh, idx)
        mv = (csh & b) != 0
        v = jnp.where(mv, cv, v)
        sh = jnp.where(mv, csh, sh)
    return v


def _make_sc_agg(E):
    Ew = E // NSUB
    NF = Ew // 16
    SEL = Ew + K + 16
    mesh = plsc.VectorSubcoreMesh(core_axis_name="c", subcore_axis_name="s")

    @functools.partial(
        pl.kernel,
        out_type=(jax.ShapeDtypeStruct((NDP, 2, DW), jnp.float32),
                  jax.ShapeDtypeStruct((NDP, DW), jnp.float32)),
        mesh=mesh,
        scratch_types=[
            pltpu.VMEM((Ew + 16,), jnp.int32),
            pltpu.VMEM((SEL,), jnp.int32),
            pltpu.VMEM((K,), jnp.int32),
            pltpu.VMEM((K,), jnp.int32),
            pltpu.VMEM((K, 2, DW), jnp.float32),
            pltpu.VMEM((K, DW), jnp.float32),
            pltpu.VMEM_SHARED((CP, 2, DW), jnp.float32),
            pltpu.VMEM_SHARED((CP, DW), jnp.float32),
            pltpu.SemaphoreType.DMA,
        ],
    )
    def sc_agg(ed_hbm, h_hbm, zf_hbm, zd_hbm, ones_hbm,
               agg_hbm, deg_hbm,
               ed, sel, idxs, idxd, rows, ones_v, agg_s, deg_s, sem):
        c = lax.axis_index("c")
        s = lax.axis_index("s")
        pltpu.sync_copy(ed_hbm.at[pl.ds(s * Ew, Ew)], ed.at[pl.ds(0, Ew)])
        pltpu.sync_copy(ones_hbm, ones_v)

        iota = lax.iota(jnp.int32, 16)
        one = jnp.full((16,), 1, jnp.int32)
        zero = jnp.full((16,), 0, jnp.int32)
        garb = jnp.full((16,), C, jnp.int32)
        egarb = jnp.full((16,), 65535, jnp.int32)
        ar1 = iota + 1

        hlo = c * HALF

        def pfbody(i, cnt):
            pk = ed[pl.ds(i * 16, 16)]
            d = pk & (2 ** EPK - 1)
            m = (d >= hlo) & (d < hlo + HALF)
            mi = jnp.where(m, one, zero)
            cs = _prefix16(mi, iota, zero)
            sh = jnp.where(m, ar1 - cs, zero)
            v = _compact16(jnp.where(m, pk, egarb), sh, iota)
            ed[pl.ds(cnt, 16)] = v
            return cnt + cs[15]

        cnt0 = lax.fori_loop(0, NF, pfbody, jnp.int32(0), unroll=2)
        ed[pl.ds(cnt0, 16)] = egarb
        nf0 = (cnt0 + 15) // 16

        for j in range(NCH):
            lo = hlo + j * C
            pltpu.sync_copy(zf_hbm, agg_s.at[pl.ds(s * STR, STR)])
            pltpu.sync_copy(zd_hbm, deg_s.at[pl.ds(s * STR, STR)])
            plsc.subcore_barrier()

            def fbody(i, cnt):
                pk = ed[pl.ds(i * 16, 16)]
                d = pk & (2 ** EPK - 1)
                m = (d >= lo) & (d < lo + C)
                mi = jnp.where(m, one, zero)
                cs = _prefix16(mi, iota, zero)
                sh = jnp.where(m, ar1 - cs, zero)
                sv = lax.shift_right_logical(pk, EPK)
                pkc = jnp.where(m, lax.shift_left(sv, PK) + (d - lo), garb)
                sel[pl.ds(cnt, 16)] = _compact16(pkc, sh, iota)
                return cnt + cs[15]

            cnt = lax.fori_loop(0, nf0, fbody, jnp.int32(0))

            def pbody(u, acc):
                sel[pl.ds(cnt + u * 16, 16)] = garb
                return acc

            lax.fori_loop(0, K // 16, pbody, jnp.int32(0))

            def bbody(b, acc):
                for u in range(K // 16):
                    pkv = sel[pl.ds(b * K + u * 16, 16)]
                    idxs[pl.ds(u * 16, 16)] = lax.shift_right_logical(pkv, PK)
                    idxd[pl.ds(u * 16, 16)] = pkv & (2 ** PK - 1)

                pltpu.async_copy(h_hbm.at[idxs], rows, sem).wait()
                pltpu.sync_copy(rows, agg_s.at[idxd], add=True)
                pltpu.sync_copy(ones_v, deg_s.at[idxd], add=True)
                return acc

            lax.fori_loop(0, (cnt + (K - 1)) // K, bbody, jnp.int32(0))
            plsc.subcore_barrier()

            TAIL = C - (NSUB - 1) * STR

            @pl.when(s < NSUB - 1)
            def _wb():
                pltpu.sync_copy(agg_s.at[pl.ds(s * STR, STR)],
                                agg_hbm.at[pl.ds(lo + s * STR, STR)])
                pltpu.sync_copy(deg_s.at[pl.ds(s * STR, STR)],
                                deg_hbm.at[pl.ds(lo + s * STR, STR)])

            @pl.when(s == NSUB - 1)
            def _wbt():
                pltpu.sync_copy(agg_s.at[pl.ds((NSUB - 1) * STR, TAIL)],
                                agg_hbm.at[pl.ds(lo + (NSUB - 1) * STR, TAIL)])
                pltpu.sync_copy(deg_s.at[pl.ds((NSUB - 1) * STR, TAIL)],
                                deg_hbm.at[pl.ds(lo + (NSUB - 1) * STR, TAIL)])

    return sc_agg


_sc_agg_1 = _make_sc_agg(E1)
_sc_agg_2 = _make_sc_agg(E2)

BT = 1000


def _tc_body(agg_ref, deg_ref, hdst_ref, m_ref, w_ref, b_ref, out_ref):
    deg = jnp.maximum(deg_ref[:, 0:1], 1.0)
    aggn = agg_ref[...] / deg
    wa = w_ref[:, 0:F]
    wh = w_ref[:, F:2 * F]
    dn = (((1,), (1,)), ((), ()))
    acc = lax.dot_general(aggn, wa, dn, preferred_element_type=jnp.float32,
                          precision=lax.Precision.HIGHEST)
    acc = acc + lax.dot_general(hdst_ref[...], wh, dn,
                                preferred_element_type=jnp.float32,
                                precision=lax.Precision.HIGHEST)
    acc = jnp.maximum(acc + b_ref[...], 0.0)
    out_ref[...] = jnp.where(m_ref[...] != 0, acc * 2.0, 0.0)


def _tc_layer(agg, deg, hdst, mask_i8, W, b):
    return pl.pallas_call(
        _tc_body,
        grid=(ND // BT,),
        in_specs=[
            pl.BlockSpec((BT, F), lambda i: (i, 0)),
            pl.BlockSpec((BT, DW), lambda i: (i, 0)),
            pl.BlockSpec((BT, F), lambda i: (i, 0)),
            pl.BlockSpec((BT, F), lambda i: (i, 0)),
            pl.BlockSpec((F, 2 * F), lambda i: (0, 0)),
            pl.BlockSpec((1, F), lambda i: (0, 0)),
        ],
        out_specs=pl.BlockSpec((BT, F), lambda i: (i, 0)),
        out_shape=jax.ShapeDtypeStruct((ND, F), jnp.float32),
    )(agg, deg, hdst, mask_i8, W, b)


def _rotl(x, r):
    return ((x << np.uint32(r)) | (x >> np.uint32(32 - r))).astype(np.uint32)


def _threefry2x32(k0, k1, x0, x1):
    rot = ((13, 15, 26, 6), (17, 29, 16, 24))
    ks = (np.uint32(k0), np.uint32(k1),
          np.uint32(k0) ^ np.uint32(k1) ^ np.uint32(0x1BD11BDA))
    x0 = (x0 + ks[0]).astype(np.uint32)
    x1 = (x1 + ks[1]).astype(np.uint32)
    for i in range(5):
        for r in rot[i % 2]:
            x0 = (x0 + x1).astype(np.uint32)
            x1 = _rotl(x1, r)
            x1 = x1 ^ x0
        x0 = (x0 + ks[(i + 1) % 3]).astype(np.uint32)
        x1 = (x1 + ks[(i + 2) % 3] + np.uint32(i + 1)).astype(np.uint32)
    return x0, x1


def _np_split(k0, k1):
    a, b = _threefry2x32(k0, k1, np.zeros(2, np.uint32),
                         np.arange(2, dtype=np.uint32))
    return (a[0], b[0]), (a[1], b[1])


def _np_bernoulli_half(k0, k1, shape):
    n = int(np.prod(shape))
    a, b = _threefry2x32(k0, k1, np.zeros(n, np.uint32),
                         np.arange(n, dtype=np.uint32))
    w = a ^ b
    fl = (((w >> np.uint32(9)) | np.uint32(0x3F800000)).view(np.float32)
          - np.float32(1.0))
    return (fl < np.float32(0.5)).reshape(shape)


def _dropout_masks():
    knew, sub = _np_split(np.uint32(0), np.uint32(42))
    m1 = _np_bernoulli_half(sub[0], sub[1], (50000, F))[:ND]
    _, sub2 = _np_split(*knew)
    m2 = _np_bernoulli_half(sub2[0], sub2[1], (ND, F))
    return m1.astype(np.int8), m2.astype(np.int8)


_M1, _M2 = _dropout_masks()


def kernel(x, edge, num_sampled_nodes, num_sampled_edges, W1, b1, W2, b2):
    del num_sampled_nodes, num_sampled_edges
    edge = edge.astype(jnp.int32)
    ed1 = (edge[E2:, 0] << EPK) | edge[E2:, 1]
    ed2 = (edge[:E2, 0] << EPK) | edge[:E2, 1]

    m1, m2 = _M1, _M2

    zf = jnp.zeros((STR, 2, DW), jnp.float32)
    zd = jnp.zeros((STR, DW), jnp.float32)
    ones = jnp.ones((K, DW), jnp.float32)

    agg1, deg1 = _sc_agg_1(ed1, x.reshape(-1, 2, DW), zf, zd, ones)
    agg1 = agg1.reshape(-1, F)
    h1 = _tc_layer(agg1, deg1, x, m1, W1, b1.reshape(1, F))
    agg2, deg2 = _sc_agg_2(ed2, h1.reshape(-1, 2, DW), zf, zd, ones)
    agg2 = agg2.reshape(-1, F)
    return _tc_layer(agg2, deg2, h1, m2, W2, b2.reshape(1, F))
